# baseline (speedup 1.0000x reference)
import jax
import jax.numpy as jnp
from jax import lax
from jax.experimental import pallas as pl
from jax.experimental.pallas import tpu as pltpu
from jax.experimental.pallas import tpu_sc as plsc

N = 10000
D = 128
E = 320000

NC = 2
NS = 16
NW = NC * NS
CH = 128
NPAD = 10240
RPT = NPAD // NS
ECH = 2560
EPAD = ECH * CH
CPT = ECH // NW

NB = 4
NG = CPT // NB
GS = 8
NGRP = CPT // GS

CS_ROW = 32
CF_ROW = (ECH - NS * CS_ROW) // NS
SLOW_CID = 1
ROW_SPLIT = NS * CS_ROW

_f32 = jnp.float32


def _widx(cid, sid):
    return cid * NS + sid


def _zero16():
    return jnp.zeros((16,), _f32)


def _deg_body(dst_hbm, out_hbm, dstv, ones_v, zb_v, acc_sp):
    cid = lax.axis_index("c")
    sid = lax.axis_index("s")
    wid = _widx(cid, sid)
    z16 = _zero16()
    o16 = jnp.ones((16,), _f32)
    for i in range(CH // 16):
        ones_v[pl.ds(i * 16, 16)] = o16
        zb_v[pl.ds(i * 16, 16)] = z16
    for k in range(RPT // CH):
        pltpu.sync_copy(zb_v, acc_sp.at[pl.ds(sid * RPT + k * CH, CH)])
    pltpu.sync_copy(dst_hbm.at[pl.ds(wid * CPT, CPT)], dstv)
    plsc.subcore_barrier()

    def chunk(j, c):
        pltpu.sync_copy(ones_v, acc_sp.at[dstv.at[j]], add=True)
        return c

    lax.fori_loop(0, CPT, chunk, 0)
    plsc.subcore_barrier()
    pltpu.sync_copy(acc_sp.at[pl.ds(sid * RPT, RPT)],
                    out_hbm.at[cid, pl.ds(sid * RPT, RPT)])


def _row_body(y_hbm, src_hbm, dst_hbm, out_hbm,
              sidx0, sidx1, didx0, didx1, r0, r1,
              gs0, gs1, is0, is1, acc_sp):
    bufs = (r0, r1)
    gsem = (gs0, gs1)
    sidx = (sidx0, sidx1)
    didx = (didx0, didx1)
    isem = (is0, is1)
    cid = lax.axis_index("c")
    sid = lax.axis_index("s")
    wid = _widx(cid, sid)
    z16 = _zero16()

    def zrow(r, c):
        for l in range(D // 16):
            r0[r, pl.ds(l * 16, 16)] = z16
        return c

    lax.fori_loop(0, CH, zrow, 0)
    for k in range(RPT // CH):
        pltpu.sync_copy(r0, acc_sp.at[pl.ds(sid * RPT + k * CH, CH)])
    plsc.subcore_barrier()

    slow = (cid == SLOW_CID)
    ebase = jnp.where(slow, sid * CS_ROW, ROW_SPLIT + sid * CF_ROW)
    ngrp = jnp.where(slow, CS_ROW // GS, CF_ROW // GS)
    pltpu.sync_copy(src_hbm.at[pl.ds(ebase, GS)], sidx0)
    pltpu.sync_copy(dst_hbm.at[pl.ds(ebase, GS)], didx0)
    pltpu.async_copy(src_hbm.at[pl.ds(ebase + GS, GS)], sidx1, is1)
    pltpu.async_copy(dst_hbm.at[pl.ds(ebase + GS, GS)], didx1, is1)
    pltpu.async_copy(y_hbm.at[sidx0.at[0]], r0, gs0)
    pltpu.async_copy(y_hbm.at[sidx0.at[1]], r1, gs1)

    def _wait_idx(p):
        pltpu.make_async_copy(src_hbm.at[pl.ds(0, GS)], sidx[p], isem[p]).wait()
        pltpu.make_async_copy(dst_hbm.at[pl.ds(0, GS)], didx[p], isem[p]).wait()

    def pair(q, c):
        for par in range(2):
            g = 2 * q + par
            pn = 1 - par
            for b8 in range(GS):
                bp = b8 % 2
                pltpu.make_async_copy(y_hbm.at[sidx[par].at[0]],
                                      bufs[bp], gsem[bp]).wait()
                pltpu.sync_copy(bufs[bp], acc_sp.at[didx[par].at[b8]], add=True)
                if b8 < GS - 2:
                    pltpu.async_copy(y_hbm.at[sidx[par].at[b8 + 2]],
                                     bufs[bp], gsem[bp])
                else:
                    if b8 == GS - 2:
                        @pl.when(g < ngrp - 1)
                        def _():
                            _wait_idx(pn)

                    @pl.when(g < ngrp - 1)
                    def _():
                        pltpu.async_copy(y_hbm.at[sidx[pn].at[b8 - (GS - 2)]],
                                         bufs[bp], gsem[bp])

            @pl.when(g + 2 < ngrp)
            def _():
                off = ebase + (g + 2) * GS
                pltpu.async_copy(src_hbm.at[pl.ds(off, GS)], sidx[par],
                                 isem[par])
                pltpu.async_copy(dst_hbm.at[pl.ds(off, GS)], didx[par],
                                 isem[par])

        return c

    lax.fori_loop(0, ngrp // 2, pair, 0)
    plsc.subcore_barrier()
    pltpu.sync_copy(acc_sp.at[pl.ds(sid * RPT, RPT)],
                    out_hbm.at[cid, pl.ds(sid * RPT, RPT)])


def _sca_body(t_hbm, src_hbm, dst_hbm, out_hbm, srcv, dstv,
              v0, v1, v2, v3, g0, g1, g2, g3, t_sp, acc_sp):
    bufs = (v0, v1, v2, v3)
    sems = (g0, g1, g2, g3)
    cid = lax.axis_index("c")
    sid = lax.axis_index("s")
    wid = _widx(cid, sid)
    z16 = _zero16()
    for i in range(CH // 16):
        v0[pl.ds(i * 16, 16)] = z16
    for k in range(RPT // CH):
        pltpu.sync_copy(v0, acc_sp.at[pl.ds(sid * RPT + k * CH, CH)])

    @pl.when(sid == 0)
    def _():
        pltpu.sync_copy(t_hbm, t_sp)

    pltpu.sync_copy(src_hbm.at[pl.ds(wid * CPT, CPT)], srcv)
    pltpu.sync_copy(dst_hbm.at[pl.ds(wid * CPT, CPT)], dstv)
    plsc.subcore_barrier()

    for b in range(NB):
        pltpu.async_copy(t_sp.at[srcv.at[b]], bufs[b], sems[b])

    def group(m, c):
        for b in range(NB):
            j = m * NB + b
            pltpu.make_async_copy(t_sp.at[srcv.at[0]], bufs[b], sems[b]).wait()
            pltpu.sync_copy(bufs[b], acc_sp.at[dstv.at[j]], add=True)

            @pl.when(m < NG - 1)
            def _():
                pltpu.async_copy(t_sp.at[srcv.at[j + NB]], bufs[b], sems[b])

        return c

    lax.fori_loop(0, NG, group, 0)
    plsc.subcore_barrier()
    pltpu.sync_copy(acc_sp.at[pl.ds(sid * RPT, RPT)],
                    out_hbm.at[cid, pl.ds(sid * RPT, RPT)])


BLK = 1000
GRD = N // BLK


def _tc1_body(x_ref, w1_ref, d0_ref, d1_ref, ydup_ref, dis_ref):
    deg = d0_ref[...] + d1_ref[...] + 1.0
    dis = lax.rsqrt(deg)
    xw = jnp.dot(x_ref[...], w1_ref[...], preferred_element_type=_f32)
    y = xw * dis
    ydup_ref[0] = y
    ydup_ref[1] = y
    dis_ref[...] = dis


def _sigmoid(u):
    return 1.0 / (1.0 + jnp.exp(-u))


def _tc2_body(a0_ref, a1_ref, y_ref, dis_ref, b1_ref, w2_ref, t_ref):
    dis = dis_ref[...]
    z = a0_ref[...] + a1_ref[...] + y_ref[...]
    x1 = _sigmoid(dis * z + b1_ref[...][None, :])
    tcol = jnp.dot(x1, w2_ref[...], preferred_element_type=_f32)
    t_ref[...] = tcol * dis


def _tc3_body(q0_ref, q1_ref, t_ref, dis_ref, b2_ref, o_ref):
    dis = dis_ref[...]
    g = dis * (q0_ref[...] + q1_ref[...] + t_ref[...]) + b2_ref[...]
    o_ref[...] = _sigmoid(g)


def _col_spec(i):
    return pl.BlockSpec((BLK, 1), lambda i: (i, 0))


def kernel(x, pos_edge_index, edge_attr, W1, b1, W2, b2):
    del edge_attr
    src = pos_edge_index[0]
    dst = pos_edge_index[1]
    npad = EPAD - E
    srcp = jnp.concatenate([src, jnp.zeros((npad,), jnp.int32)]).reshape(ECH, CH)
    pad_rows = N + jnp.arange(npad, dtype=jnp.int32) % (NPAD - N)
    dstp = jnp.concatenate([dst, pad_rows]).reshape(ECH, CH)
    srcp2 = srcp + jnp.where(
        jnp.arange(ECH, dtype=jnp.int32)[:, None] >= ROW_SPLIT,
        jnp.int32(N), jnp.int32(0))

    mesh = plsc.VectorSubcoreMesh(core_axis_name="c", subcore_axis_name="s",
                                  num_cores=NC, num_subcores=NS)

    deg_parts = pl.kernel(
        _deg_body,
        out_type=jax.ShapeDtypeStruct((NC, NPAD), _f32),
        mesh=mesh,
        scratch_types=[
            pltpu.VMEM((CPT, CH), jnp.int32),
            pltpu.VMEM((CH,), _f32),
            pltpu.VMEM((CH,), _f32),
            pltpu.VMEM_SHARED((NPAD,), _f32),
        ],
    )(dstp)

    d0 = deg_parts[0, :N, None]
    d1 = deg_parts[1, :N, None]

    ydup, dis = pl.pallas_call(
        _tc1_body,
        grid=(GRD,),
        in_specs=[
            pl.BlockSpec((BLK, D), lambda i: (i, 0)),
            pl.BlockSpec((D, D), lambda i: (0, 0)),
            _col_spec(0),
            _col_spec(0),
        ],
        out_specs=[
            pl.BlockSpec((2, BLK, D), lambda i: (0, i, 0)),
            _col_spec(0),
        ],
        out_shape=[
            jax.ShapeDtypeStruct((2, N, D), _f32),
            jax.ShapeDtypeStruct((N, 1), _f32),
        ],
    )(x, W1, d0, d1)

    y2x = jnp.reshape(ydup, (2 * N, D))

    acc_parts = pl.kernel(
        _row_body,
        out_type=jax.ShapeDtypeStruct((NC, NPAD, D), _f32),
        mesh=mesh,
        scratch_types=[
            pltpu.VMEM((GS, CH), jnp.int32),
            pltpu.VMEM((GS, CH), jnp.int32),
            pltpu.VMEM((GS, CH), jnp.int32),
            pltpu.VMEM((GS, CH), jnp.int32),
            pltpu.VMEM((CH, D), _f32),
            pltpu.VMEM((CH, D), _f32),
            pltpu.SemaphoreType.DMA,
            pltpu.SemaphoreType.DMA,
            pltpu.SemaphoreType.DMA,
            pltpu.SemaphoreType.DMA,
            pltpu.VMEM_SHARED((NPAD, D), _f32),
        ],
    )(y2x, srcp2, dstp)

    t = pl.pallas_call(
        _tc2_body,
        grid=(GRD,),
        in_specs=[
            pl.BlockSpec((BLK, D), lambda i: (i, 0)),
            pl.BlockSpec((BLK, D), lambda i: (i, 0)),
            pl.BlockSpec((BLK, D), lambda i: (i, 0)),
            _col_spec(0),
            pl.BlockSpec((D,), lambda i: (0,)),
            pl.BlockSpec((D, 1), lambda i: (0, 0)),
        ],
        out_specs=_col_spec(0),
        out_shape=jax.ShapeDtypeStruct((N, 1), _f32),
    )(acc_parts[0, :N], acc_parts[1, :N], y2x[:N], dis, b1, W2)

    q_parts = pl.kernel(
        _sca_body,
        out_type=jax.ShapeDtypeStruct((NC, NPAD), _f32),
        mesh=mesh,
        scratch_types=[
            pltpu.VMEM((CPT, CH), jnp.int32),
            pltpu.VMEM((CPT, CH), jnp.int32),
            pltpu.VMEM((CH,), _f32),
            pltpu.VMEM((CH,), _f32),
            pltpu.VMEM((CH,), _f32),
            pltpu.VMEM((CH,), _f32),
            pltpu.SemaphoreType.DMA,
            pltpu.SemaphoreType.DMA,
            pltpu.SemaphoreType.DMA,
            pltpu.SemaphoreType.DMA,
            pltpu.VMEM_SHARED((N,), _f32),
            pltpu.VMEM_SHARED((NPAD,), _f32),
        ],
    )(jnp.reshape(t, (N,)), srcp, dstp)

    out = pl.pallas_call(
        _tc3_body,
        grid=(GRD,),
        in_specs=[
            _col_spec(0),
            _col_spec(0),
            _col_spec(0),
            _col_spec(0),
            pl.BlockSpec((1, 1), lambda i: (0, 0)),
        ],
        out_specs=_col_spec(0),
        out_shape=jax.ShapeDtypeStruct((N, 1), _f32),
    )(q_parts[0, :N, None], q_parts[1, :N, None], t, dis, b2[:, None])

    return out


# speedup vs baseline: 1.1294x; 1.1294x over previous
import jax
import jax.numpy as jnp
from jax import lax
from jax.experimental import pallas as pl
from jax.experimental.pallas import tpu as pltpu
from jax.experimental.pallas import tpu_sc as plsc

N = 10000
D = 128
E = 320000

NC = 2
NS = 16
NW = NC * NS
CH = 128
NPAD = 10240
RPT = NPAD // NS
ECH = 2560
EPAD = ECH * CH
CPT = ECH // NW

NB = 4
NG = CPT // NB
GS = 8
NGRP = CPT // GS

_f32 = jnp.float32


def _widx(cid, sid):
    return cid * NS + sid


def _zero16():
    return jnp.zeros((16,), _f32)


def _deg_body(dst_hbm, out_hbm, dstv, ones_v, zb_v, acc_sp):
    cid = lax.axis_index("c")
    sid = lax.axis_index("s")
    wid = _widx(cid, sid)
    z16 = _zero16()
    o16 = jnp.ones((16,), _f32)
    for i in range(CH // 16):
        ones_v[pl.ds(i * 16, 16)] = o16
        zb_v[pl.ds(i * 16, 16)] = z16
    for k in range(RPT // CH):
        pltpu.sync_copy(zb_v, acc_sp.at[pl.ds(sid * RPT + k * CH, CH)])
    pltpu.sync_copy(dst_hbm.at[pl.ds(wid * CPT, CPT)], dstv)
    plsc.subcore_barrier()

    def chunk(j, c):
        pltpu.sync_copy(ones_v, acc_sp.at[dstv.at[j]], add=True)
        return c

    lax.fori_loop(0, CPT, chunk, 0)
    plsc.subcore_barrier()
    pltpu.sync_copy(acc_sp.at[pl.ds(sid * RPT, RPT)],
                    out_hbm.at[cid, pl.ds(sid * RPT, RPT)])


def _row_body(y_hbm, src_hbm, dst_hbm, out_hbm,
              sidx0, sidx1, didx0, didx1, r0, r1,
              gs0, gs1, is0, is1, acc_sp):
    bufs = (r0, r1)
    gsem = (gs0, gs1)
    sidx = (sidx0, sidx1)
    didx = (didx0, didx1)
    isem = (is0, is1)
    cid = lax.axis_index("c")
    sid = lax.axis_index("s")
    wid = _widx(cid, sid)
    z16 = _zero16()

    def zrow(r, c):
        for l in range(D // 16):
            r0[r, pl.ds(l * 16, 16)] = z16
        return c

    lax.fori_loop(0, CH, zrow, 0)
    for k in range(RPT // CH):
        pltpu.sync_copy(r0, acc_sp.at[pl.ds(sid * RPT + k * CH, CH)])
    plsc.subcore_barrier()

    ebase = wid * CPT
    pltpu.sync_copy(src_hbm.at[pl.ds(ebase, GS)], sidx0)
    pltpu.sync_copy(dst_hbm.at[pl.ds(ebase, GS)], didx0)
    pltpu.async_copy(src_hbm.at[pl.ds(ebase + GS, GS)], sidx1, is1)
    pltpu.async_copy(dst_hbm.at[pl.ds(ebase + GS, GS)], didx1, is1)
    pltpu.async_copy(y_hbm.at[sidx0.at[0]], r0, gs0)
    pltpu.async_copy(y_hbm.at[sidx0.at[1]], r1, gs1)

    def _wait_idx(p):
        pltpu.make_async_copy(src_hbm.at[pl.ds(0, GS)], sidx[p], isem[p]).wait()
        pltpu.make_async_copy(dst_hbm.at[pl.ds(0, GS)], didx[p], isem[p]).wait()

    def pair(q, c):
        for par in range(2):
            g = 2 * q + par
            pn = 1 - par
            for b8 in range(GS):
                bp = b8 % 2
                pltpu.make_async_copy(y_hbm.at[sidx[par].at[0]],
                                      bufs[bp], gsem[bp]).wait()
                pltpu.sync_copy(bufs[bp], acc_sp.at[didx[par].at[b8]], add=True)
                if b8 < GS - 2:
                    pltpu.async_copy(y_hbm.at[sidx[par].at[b8 + 2]],
                                     bufs[bp], gsem[bp])
                else:
                    if b8 == GS - 2:
                        @pl.when(g < NGRP - 1)
                        def _():
                            _wait_idx(pn)

                    @pl.when(g < NGRP - 1)
                    def _():
                        pltpu.async_copy(y_hbm.at[sidx[pn].at[b8 - (GS - 2)]],
                                         bufs[bp], gsem[bp])

            @pl.when(g + 2 < NGRP)
            def _():
                off = ebase + (g + 2) * GS
                pltpu.async_copy(src_hbm.at[pl.ds(off, GS)], sidx[par],
                                 isem[par])
                pltpu.async_copy(dst_hbm.at[pl.ds(off, GS)], didx[par],
                                 isem[par])

        return c

    lax.fori_loop(0, NGRP // 2, pair, 0)
    plsc.subcore_barrier()
    pltpu.sync_copy(acc_sp.at[pl.ds(sid * RPT, RPT)],
                    out_hbm.at[cid, pl.ds(sid * RPT, RPT)])


def _sca_body(t_hbm, src_hbm, dst_hbm, out_hbm, srcv, dstv,
              v0, v1, v2, v3, g0, g1, g2, g3, t_sp, acc_sp):
    bufs = (v0, v1, v2, v3)
    sems = (g0, g1, g2, g3)
    cid = lax.axis_index("c")
    sid = lax.axis_index("s")
    wid = _widx(cid, sid)
    z16 = _zero16()
    for i in range(CH // 16):
        v0[pl.ds(i * 16, 16)] = z16
    for k in range(RPT // CH):
        pltpu.sync_copy(v0, acc_sp.at[pl.ds(sid * RPT + k * CH, CH)])

    @pl.when(sid == 0)
    def _():
        pltpu.sync_copy(t_hbm, t_sp)

    pltpu.sync_copy(src_hbm.at[pl.ds(wid * CPT, CPT)], srcv)
    pltpu.sync_copy(dst_hbm.at[pl.ds(wid * CPT, CPT)], dstv)
    plsc.subcore_barrier()

    for b in range(NB):
        pltpu.async_copy(t_sp.at[srcv.at[b]], bufs[b], sems[b])

    def group(m, c):
        for b in range(NB):
            j = m * NB + b
            pltpu.make_async_copy(t_sp.at[srcv.at[0]], bufs[b], sems[b]).wait()
            pltpu.sync_copy(bufs[b], acc_sp.at[dstv.at[j]], add=True)

            @pl.when(m < NG - 1)
            def _():
                pltpu.async_copy(t_sp.at[srcv.at[j + NB]], bufs[b], sems[b])

        return c

    lax.fori_loop(0, NG, group, 0)
    plsc.subcore_barrier()
    pltpu.sync_copy(acc_sp.at[pl.ds(sid * RPT, RPT)],
                    out_hbm.at[cid, pl.ds(sid * RPT, RPT)])


BLK = 1000
GRD = N // BLK


def _tc1_body(x_ref, w1_ref, d0_ref, d1_ref, ydup_ref, dis_ref):
    deg = d0_ref[...] + d1_ref[...] + 1.0
    dis = lax.rsqrt(deg)
    xw = jnp.dot(x_ref[...], w1_ref[...], preferred_element_type=_f32)
    y = xw * dis
    ydup_ref[0] = y
    ydup_ref[1] = y
    dis_ref[...] = dis


def _sigmoid(u):
    return 1.0 / (1.0 + jnp.exp(-u))


def _tc2_body(a0_ref, a1_ref, y_ref, dis_ref, b1_ref, w2_ref, t_ref):
    dis = dis_ref[...]
    z = a0_ref[...] + a1_ref[...] + y_ref[...]
    x1 = _sigmoid(dis * z + b1_ref[...][None, :])
    tcol = jnp.dot(x1, w2_ref[...], preferred_element_type=_f32)
    t_ref[...] = tcol * dis


def _tc3_body(q0_ref, q1_ref, t_ref, dis_ref, b2_ref, o_ref):
    dis = dis_ref[...]
    g = dis * (q0_ref[...] + q1_ref[...] + t_ref[...]) + b2_ref[...]
    o_ref[...] = _sigmoid(g)


def _col_spec(i):
    return pl.BlockSpec((BLK, 1), lambda i: (i, 0))


def kernel(x, pos_edge_index, edge_attr, W1, b1, W2, b2):
    del edge_attr
    src = pos_edge_index[0]
    dst = pos_edge_index[1]
    npad = EPAD - E
    srcp = jnp.concatenate([src, jnp.zeros((npad,), jnp.int32)]).reshape(ECH, CH)
    pad_rows = N + jnp.arange(npad, dtype=jnp.int32) % (NPAD - N)
    dstp = jnp.concatenate([dst, pad_rows]).reshape(ECH, CH)
    srcp2 = srcp + jnp.where(
        jnp.arange(ECH, dtype=jnp.int32)[:, None] >= ECH // 2,
        jnp.int32(N), jnp.int32(0))

    mesh = plsc.VectorSubcoreMesh(core_axis_name="c", subcore_axis_name="s",
                                  num_cores=NC, num_subcores=NS)

    deg_parts = pl.kernel(
        _deg_body,
        out_type=jax.ShapeDtypeStruct((NC, NPAD), _f32),
        mesh=mesh,
        scratch_types=[
            pltpu.VMEM((CPT, CH), jnp.int32),
            pltpu.VMEM((CH,), _f32),
            pltpu.VMEM((CH,), _f32),
            pltpu.VMEM_SHARED((NPAD,), _f32),
        ],
    )(dstp)

    d0 = deg_parts[0, :N, None]
    d1 = deg_parts[1, :N, None]

    ydup, dis = pl.pallas_call(
        _tc1_body,
        grid=(GRD,),
        in_specs=[
            pl.BlockSpec((BLK, D), lambda i: (i, 0)),
            pl.BlockSpec((D, D), lambda i: (0, 0)),
            _col_spec(0),
            _col_spec(0),
        ],
        out_specs=[
            pl.BlockSpec((2, BLK, D), lambda i: (0, i, 0)),
            _col_spec(0),
        ],
        out_shape=[
            jax.ShapeDtypeStruct((2, N, D), _f32),
            jax.ShapeDtypeStruct((N, 1), _f32),
        ],
    )(x, W1, d0, d1)

    y2x = jnp.reshape(ydup, (2 * N, D))

    acc_parts = pl.kernel(
        _row_body,
        out_type=jax.ShapeDtypeStruct((NC, NPAD, D), _f32),
        mesh=mesh,
        scratch_types=[
            pltpu.VMEM((GS, CH), jnp.int32),
            pltpu.VMEM((GS, CH), jnp.int32),
            pltpu.VMEM((GS, CH), jnp.int32),
            pltpu.VMEM((GS, CH), jnp.int32),
            pltpu.VMEM((CH, D), _f32),
            pltpu.VMEM((CH, D), _f32),
            pltpu.SemaphoreType.DMA,
            pltpu.SemaphoreType.DMA,
            pltpu.SemaphoreType.DMA,
            pltpu.SemaphoreType.DMA,
            pltpu.VMEM_SHARED((NPAD, D), _f32),
        ],
    )(y2x, srcp2, dstp)

    t = pl.pallas_call(
        _tc2_body,
        grid=(GRD,),
        in_specs=[
            pl.BlockSpec((BLK, D), lambda i: (i, 0)),
            pl.BlockSpec((BLK, D), lambda i: (i, 0)),
            pl.BlockSpec((BLK, D), lambda i: (i, 0)),
            _col_spec(0),
            pl.BlockSpec((D,), lambda i: (0,)),
            pl.BlockSpec((D, 1), lambda i: (0, 0)),
        ],
        out_specs=_col_spec(0),
        out_shape=jax.ShapeDtypeStruct((N, 1), _f32),
    )(acc_parts[0, :N], acc_parts[1, :N], y2x[:N], dis, b1, W2)

    q_parts = pl.kernel(
        _sca_body,
        out_type=jax.ShapeDtypeStruct((NC, NPAD), _f32),
        mesh=mesh,
        scratch_types=[
            pltpu.VMEM((CPT, CH), jnp.int32),
            pltpu.VMEM((CPT, CH), jnp.int32),
            pltpu.VMEM((CH,), _f32),
            pltpu.VMEM((CH,), _f32),
            pltpu.VMEM((CH,), _f32),
            pltpu.VMEM((CH,), _f32),
            pltpu.SemaphoreType.DMA,
            pltpu.SemaphoreType.DMA,
            pltpu.SemaphoreType.DMA,
            pltpu.SemaphoreType.DMA,
            pltpu.VMEM_SHARED((N,), _f32),
            pltpu.VMEM_SHARED((NPAD,), _f32),
        ],
    )(jnp.reshape(t, (N,)), srcp, dstp)

    out = pl.pallas_call(
        _tc3_body,
        grid=(GRD,),
        in_specs=[
            _col_spec(0),
            _col_spec(0),
            _col_spec(0),
            _col_spec(0),
            pl.BlockSpec((1, 1), lambda i: (0, 0)),
        ],
        out_specs=_col_spec(0),
        out_shape=jax.ShapeDtypeStruct((N, 1), _f32),
    )(q_parts[0, :N, None], q_parts[1, :N, None], t, dis, b2[:, None])

    return out


# speedup vs baseline: 1.1418x; 1.0110x over previous
import jax
import jax.numpy as jnp
from jax import lax
from jax.experimental import pallas as pl
from jax.experimental.pallas import tpu as pltpu
from jax.experimental.pallas import tpu_sc as plsc

N = 10000
D = 128
E = 320000

NC = 2
NS = 16
NW = NC * NS
CH = 128
NPAD = 10240
RPT = NPAD // NS
ECH = 2560
EPAD = ECH * CH
CPT = ECH // NW

NB = 4
NG = CPT // NB
GS = 8
NGRP = CPT // GS

_f32 = jnp.float32


def _widx(cid, sid):
    return cid * NS + sid


def _zero16():
    return jnp.zeros((16,), _f32)


def _deg_body(dst_hbm, out_hbm, dstv, ones_v, zb_v, acc_sp):
    cid = lax.axis_index("c")
    sid = lax.axis_index("s")
    wid = _widx(cid, sid)
    z16 = _zero16()
    o16 = jnp.ones((16,), _f32)
    for i in range(CH // 16):
        ones_v[pl.ds(i * 16, 16)] = o16
        zb_v[pl.ds(i * 16, 16)] = z16
    for k in range(RPT // CH):
        pltpu.sync_copy(zb_v, acc_sp.at[pl.ds(sid * RPT + k * CH, CH)])
    pltpu.sync_copy(dst_hbm.at[pl.ds(wid * CPT, CPT)], dstv)
    plsc.subcore_barrier()

    def chunk(j, c):
        pltpu.sync_copy(ones_v, acc_sp.at[dstv.at[j]], add=True)
        return c

    lax.fori_loop(0, CPT, chunk, 0)
    plsc.subcore_barrier()
    pltpu.sync_copy(acc_sp.at[pl.ds(sid * RPT, RPT)],
                    out_hbm.at[cid, pl.ds(sid * RPT, RPT)])


def _row_body(y_hbm, src_hbm, dst_hbm, out_hbm,
              sidx0, sidx1, didx0, didx1, r0, r1,
              gs0, gs1, is0, is1, acc_sp):
    bufs = (r0, r1)
    gsem = (gs0, gs1)
    sidx = (sidx0, sidx1)
    didx = (didx0, didx1)
    isem = (is0, is1)
    cid = lax.axis_index("c")
    sid = lax.axis_index("s")
    wid = _widx(cid, sid)
    z16 = _zero16()

    def zrow(r, c):
        for l in range(D // 16):
            r0[r, pl.ds(l * 16, 16)] = z16
        return c

    lax.fori_loop(0, CH, zrow, 0)
    for k in range(RPT // CH):
        pltpu.sync_copy(r0, acc_sp.at[pl.ds(sid * RPT + k * CH, CH)])
    plsc.subcore_barrier()

    ebase = wid * CPT
    pltpu.sync_copy(src_hbm.at[pl.ds(ebase, GS)], sidx0)
    pltpu.sync_copy(dst_hbm.at[pl.ds(ebase, GS)], didx0)
    pltpu.async_copy(src_hbm.at[pl.ds(ebase + GS, GS)], sidx1, is1)
    pltpu.async_copy(dst_hbm.at[pl.ds(ebase + GS, GS)], didx1, is1)
    pltpu.async_copy(y_hbm.at[sidx0.at[0]], r0, gs0)
    pltpu.async_copy(y_hbm.at[sidx0.at[1]], r1, gs1)

    def _wait_idx(p):
        pltpu.make_async_copy(src_hbm.at[pl.ds(0, GS)], sidx[p], isem[p]).wait()
        pltpu.make_async_copy(dst_hbm.at[pl.ds(0, GS)], didx[p], isem[p]).wait()

    def pair(q, c):
        for par in range(2):
            g = 2 * q + par
            pn = 1 - par
            for b8 in range(GS):
                bp = b8 % 2
                pltpu.make_async_copy(y_hbm.at[sidx[par].at[0]],
                                      bufs[bp], gsem[bp]).wait()
                pltpu.sync_copy(bufs[bp], acc_sp.at[didx[par].at[b8]], add=True)
                if b8 < GS - 2:
                    pltpu.async_copy(y_hbm.at[sidx[par].at[b8 + 2]],
                                     bufs[bp], gsem[bp])
                else:
                    if b8 == GS - 2:
                        @pl.when(g < NGRP - 1)
                        def _():
                            _wait_idx(pn)

                    @pl.when(g < NGRP - 1)
                    def _():
                        pltpu.async_copy(y_hbm.at[sidx[pn].at[b8 - (GS - 2)]],
                                         bufs[bp], gsem[bp])

            @pl.when(g + 2 < NGRP)
            def _():
                off = ebase + (g + 2) * GS
                pltpu.async_copy(src_hbm.at[pl.ds(off, GS)], sidx[par],
                                 isem[par])
                pltpu.async_copy(dst_hbm.at[pl.ds(off, GS)], didx[par],
                                 isem[par])

        return c

    lax.fori_loop(0, NGRP // 2, pair, 0)
    plsc.subcore_barrier()
    pltpu.sync_copy(acc_sp.at[pl.ds(sid * RPT, RPT)],
                    out_hbm.at[cid, pl.ds(sid * RPT, RPT)])


def _sca_body(t_hbm, src_hbm, dst_hbm, out_hbm, srcv, dstv,
              v0, v1, v2, v3, g0, g1, g2, g3, t_sp, acc_sp):
    bufs = (v0, v1, v2, v3)
    sems = (g0, g1, g2, g3)
    cid = lax.axis_index("c")
    sid = lax.axis_index("s")
    wid = _widx(cid, sid)
    z16 = _zero16()
    for i in range(CH // 16):
        v0[pl.ds(i * 16, 16)] = z16
    for k in range(RPT // CH):
        pltpu.sync_copy(v0, acc_sp.at[pl.ds(sid * RPT + k * CH, CH)])

    @pl.when(sid == 0)
    def _():
        pltpu.sync_copy(t_hbm, t_sp)

    pltpu.sync_copy(src_hbm.at[pl.ds(wid * CPT, CPT)], srcv)
    pltpu.sync_copy(dst_hbm.at[pl.ds(wid * CPT, CPT)], dstv)
    plsc.subcore_barrier()

    for b in range(NB):
        pltpu.async_copy(t_sp.at[srcv.at[b]], bufs[b], sems[b])

    def group(m, c):
        for b in range(NB):
            j = m * NB + b
            pltpu.make_async_copy(t_sp.at[srcv.at[0]], bufs[b], sems[b]).wait()
            pltpu.sync_copy(bufs[b], acc_sp.at[dstv.at[j]], add=True)

            @pl.when(m < NG - 1)
            def _():
                pltpu.async_copy(t_sp.at[srcv.at[j + NB]], bufs[b], sems[b])

        return c

    lax.fori_loop(0, NG, group, 0)
    plsc.subcore_barrier()
    pltpu.sync_copy(acc_sp.at[pl.ds(sid * RPT, RPT)],
                    out_hbm.at[cid, pl.ds(sid * RPT, RPT)])


BLK = 2000
GRD = N // BLK


def _tc1_body(x_ref, w1_ref, d0_ref, d1_ref, ydup_ref, dis_ref):
    deg = d0_ref[...] + d1_ref[...] + 1.0
    dis = lax.rsqrt(deg)
    xw = jnp.dot(x_ref[...], w1_ref[...], preferred_element_type=_f32)
    y = xw * dis
    ydup_ref[0] = y
    ydup_ref[1] = y
    dis_ref[...] = dis


def _sigmoid(u):
    return 1.0 / (1.0 + jnp.exp(-u))


def _tc2_body(a0_ref, a1_ref, y_ref, dis_ref, b1_ref, w2_ref, t_ref):
    dis = dis_ref[...]
    z = a0_ref[...] + a1_ref[...] + y_ref[...]
    x1 = _sigmoid(dis * z + b1_ref[...][None, :])
    tcol = jnp.dot(x1, w2_ref[...], preferred_element_type=_f32)
    t_ref[...] = tcol * dis


def _tc3_body(q0_ref, q1_ref, t_ref, dis_ref, b2_ref, o_ref):
    dis = dis_ref[...]
    g = dis * (q0_ref[...] + q1_ref[...] + t_ref[...]) + b2_ref[...]
    o_ref[...] = _sigmoid(g)


def _col_spec(i):
    return pl.BlockSpec((BLK, 1), lambda i: (i, 0))


def kernel(x, pos_edge_index, edge_attr, W1, b1, W2, b2):
    del edge_attr
    src = pos_edge_index[0]
    dst = pos_edge_index[1]
    npad = EPAD - E
    srcp = jnp.concatenate([src, jnp.zeros((npad,), jnp.int32)]).reshape(ECH, CH)
    pad_rows = N + jnp.arange(npad, dtype=jnp.int32) % (NPAD - N)
    dstp = jnp.concatenate([dst, pad_rows]).reshape(ECH, CH)
    srcp2 = srcp + jnp.where(
        jnp.arange(ECH, dtype=jnp.int32)[:, None] >= ECH // 2,
        jnp.int32(N), jnp.int32(0))

    mesh = plsc.VectorSubcoreMesh(core_axis_name="c", subcore_axis_name="s",
                                  num_cores=NC, num_subcores=NS)

    deg_parts = pl.kernel(
        _deg_body,
        out_type=jax.ShapeDtypeStruct((NC, NPAD), _f32),
        mesh=mesh,
        scratch_types=[
            pltpu.VMEM((CPT, CH), jnp.int32),
            pltpu.VMEM((CH,), _f32),
            pltpu.VMEM((CH,), _f32),
            pltpu.VMEM_SHARED((NPAD,), _f32),
        ],
    )(dstp)

    d0 = deg_parts[0, :N, None]
    d1 = deg_parts[1, :N, None]

    ydup, dis = pl.pallas_call(
        _tc1_body,
        grid=(GRD,),
        in_specs=[
            pl.BlockSpec((BLK, D), lambda i: (i, 0)),
            pl.BlockSpec((D, D), lambda i: (0, 0)),
            _col_spec(0),
            _col_spec(0),
        ],
        out_specs=[
            pl.BlockSpec((2, BLK, D), lambda i: (0, i, 0)),
            _col_spec(0),
        ],
        out_shape=[
            jax.ShapeDtypeStruct((2, N, D), _f32),
            jax.ShapeDtypeStruct((N, 1), _f32),
        ],
    )(x, W1, d0, d1)

    y2x = jnp.reshape(ydup, (2 * N, D))

    acc_parts = pl.kernel(
        _row_body,
        out_type=jax.ShapeDtypeStruct((NC, NPAD, D), _f32),
        mesh=mesh,
        scratch_types=[
            pltpu.VMEM((GS, CH), jnp.int32),
            pltpu.VMEM((GS, CH), jnp.int32),
            pltpu.VMEM((GS, CH), jnp.int32),
            pltpu.VMEM((GS, CH), jnp.int32),
            pltpu.VMEM((CH, D), _f32),
            pltpu.VMEM((CH, D), _f32),
            pltpu.SemaphoreType.DMA,
            pltpu.SemaphoreType.DMA,
            pltpu.SemaphoreType.DMA,
            pltpu.SemaphoreType.DMA,
            pltpu.VMEM_SHARED((NPAD, D), _f32),
        ],
    )(y2x, srcp2, dstp)

    t = pl.pallas_call(
        _tc2_body,
        grid=(GRD,),
        in_specs=[
            pl.BlockSpec((BLK, D), lambda i: (i, 0)),
            pl.BlockSpec((BLK, D), lambda i: (i, 0)),
            pl.BlockSpec((BLK, D), lambda i: (i, 0)),
            _col_spec(0),
            pl.BlockSpec((D,), lambda i: (0,)),
            pl.BlockSpec((D, 1), lambda i: (0, 0)),
        ],
        out_specs=_col_spec(0),
        out_shape=jax.ShapeDtypeStruct((N, 1), _f32),
    )(acc_parts[0, :N], acc_parts[1, :N], y2x[:N], dis, b1, W2)

    q_parts = pl.kernel(
        _sca_body,
        out_type=jax.ShapeDtypeStruct((NC, NPAD), _f32),
        mesh=mesh,
        scratch_types=[
            pltpu.VMEM((CPT, CH), jnp.int32),
            pltpu.VMEM((CPT, CH), jnp.int32),
            pltpu.VMEM((CH,), _f32),
            pltpu.VMEM((CH,), _f32),
            pltpu.VMEM((CH,), _f32),
            pltpu.VMEM((CH,), _f32),
            pltpu.SemaphoreType.DMA,
            pltpu.SemaphoreType.DMA,
            pltpu.SemaphoreType.DMA,
            pltpu.SemaphoreType.DMA,
            pltpu.VMEM_SHARED((N,), _f32),
            pltpu.VMEM_SHARED((NPAD,), _f32),
        ],
    )(jnp.reshape(t, (N,)), srcp, dstp)

    out = pl.pallas_call(
        _tc3_body,
        grid=(GRD,),
        in_specs=[
            _col_spec(0),
            _col_spec(0),
            _col_spec(0),
            _col_spec(0),
            pl.BlockSpec((1, 1), lambda i: (0, 0)),
        ],
        out_specs=_col_spec(0),
        out_shape=jax.ShapeDtypeStruct((N, 1), _f32),
    )(q_parts[0, :N, None], q_parts[1, :N, None], t, dis, b2[:, None])

    return out


# speedup vs baseline: 1.1484x; 1.0057x over previous
import jax
import jax.numpy as jnp
from jax import lax
from jax.experimental import pallas as pl
from jax.experimental.pallas import tpu as pltpu
from jax.experimental.pallas import tpu_sc as plsc

N = 10000
D = 128
E = 320000

NC = 2
NS = 16
NW = NC * NS
CH = 128
NPAD = 10240
RPT = NPAD // NS
ECH = 2560
EPAD = ECH * CH
CPT = ECH // NW

NB = 4
NG = CPT // NB
GS = 8
NGRP = CPT // GS

_f32 = jnp.float32


def _widx(cid, sid):
    return cid * NS + sid


def _zero16():
    return jnp.zeros((16,), _f32)


def _deg_body(dst_hbm, out_hbm, dstv, ones_v, zb_v, acc_sp):
    cid = lax.axis_index("c")
    sid = lax.axis_index("s")
    wid = _widx(cid, sid)
    z16 = _zero16()
    o16 = jnp.ones((16,), _f32)
    for i in range(CH // 16):
        ones_v[pl.ds(i * 16, 16)] = o16
        zb_v[pl.ds(i * 16, 16)] = z16
    for k in range(RPT // CH):
        pltpu.sync_copy(zb_v, acc_sp.at[pl.ds(sid * RPT + k * CH, CH)])
    pltpu.sync_copy(dst_hbm.at[pl.ds(wid * CPT, CPT)], dstv)
    plsc.subcore_barrier()

    def chunk(j, c):
        pltpu.sync_copy(ones_v, acc_sp.at[dstv.at[j]], add=True)
        return c

    lax.fori_loop(0, CPT, chunk, 0)
    plsc.subcore_barrier()
    pltpu.sync_copy(acc_sp.at[pl.ds(sid * RPT, RPT)],
                    out_hbm.at[cid, pl.ds(sid * RPT, RPT)])


def _row_body(y_hbm, src_hbm, dst_hbm, out_hbm,
              sidx0, sidx1, didx0, didx1, r0, r1,
              gs0, gs1, is0, is1, acc_sp):
    bufs = (r0, r1)
    gsem = (gs0, gs1)
    sidx = (sidx0, sidx1)
    didx = (didx0, didx1)
    isem = (is0, is1)
    cid = lax.axis_index("c")
    sid = lax.axis_index("s")
    wid = _widx(cid, sid)
    z16 = _zero16()

    def zrow(r, c):
        for l in range(D // 16):
            r0[r, pl.ds(l * 16, 16)] = z16
        return c

    lax.fori_loop(0, CH, zrow, 0)
    for k in range(RPT // CH):
        pltpu.sync_copy(r0, acc_sp.at[pl.ds(sid * RPT + k * CH, CH)])
    plsc.subcore_barrier()

    ebase = wid * CPT
    pltpu.sync_copy(src_hbm.at[pl.ds(ebase, GS)], sidx0)
    pltpu.sync_copy(dst_hbm.at[pl.ds(ebase, GS)], didx0)
    pltpu.async_copy(src_hbm.at[pl.ds(ebase + GS, GS)], sidx1, is1)
    pltpu.async_copy(dst_hbm.at[pl.ds(ebase + GS, GS)], didx1, is1)
    pltpu.async_copy(y_hbm.at[sidx0.at[0]], r0, gs0)
    pltpu.async_copy(y_hbm.at[sidx0.at[1]], r1, gs1)

    def _wait_idx(p):
        pltpu.make_async_copy(src_hbm.at[pl.ds(0, GS)], sidx[p], isem[p]).wait()
        pltpu.make_async_copy(dst_hbm.at[pl.ds(0, GS)], didx[p], isem[p]).wait()

    def pair(q, c):
        for par in range(2):
            g = 2 * q + par
            pn = 1 - par
            for b8 in range(GS):
                bp = b8 % 2
                pltpu.make_async_copy(y_hbm.at[sidx[par].at[0]],
                                      bufs[bp], gsem[bp]).wait()
                pltpu.sync_copy(bufs[bp], acc_sp.at[didx[par].at[b8]], add=True)
                if b8 < GS - 2:
                    pltpu.async_copy(y_hbm.at[sidx[par].at[b8 + 2]],
                                     bufs[bp], gsem[bp])
                else:
                    if b8 == GS - 2:
                        @pl.when(g < NGRP - 1)
                        def _():
                            _wait_idx(pn)

                    @pl.when(g < NGRP - 1)
                    def _():
                        pltpu.async_copy(y_hbm.at[sidx[pn].at[b8 - (GS - 2)]],
                                         bufs[bp], gsem[bp])

            @pl.when(g + 2 < NGRP)
            def _():
                off = ebase + (g + 2) * GS
                pltpu.async_copy(src_hbm.at[pl.ds(off, GS)], sidx[par],
                                 isem[par])
                pltpu.async_copy(dst_hbm.at[pl.ds(off, GS)], didx[par],
                                 isem[par])

        return c

    lax.fori_loop(0, NGRP // 2, pair, 0)
    plsc.subcore_barrier()
    pltpu.sync_copy(acc_sp.at[pl.ds(sid * RPT, RPT)],
                    out_hbm.at[cid, pl.ds(sid * RPT, RPT)])


def _sca_body(t_hbm, src_hbm, dst_hbm, out_hbm, srcv, dstv,
              v0, v1, v2, v3, g0, g1, g2, g3, t_sp, acc_sp):
    bufs = (v0, v1, v2, v3)
    sems = (g0, g1, g2, g3)
    cid = lax.axis_index("c")
    sid = lax.axis_index("s")
    wid = _widx(cid, sid)
    z16 = _zero16()
    for i in range(CH // 16):
        v0[pl.ds(i * 16, 16)] = z16
    for k in range(RPT // CH):
        pltpu.sync_copy(v0, acc_sp.at[pl.ds(sid * RPT + k * CH, CH)])

    @pl.when(sid == 0)
    def _():
        pltpu.sync_copy(t_hbm, t_sp)

    pltpu.sync_copy(src_hbm.at[pl.ds(wid * CPT, CPT)], srcv)
    pltpu.sync_copy(dst_hbm.at[pl.ds(wid * CPT, CPT)], dstv)
    plsc.subcore_barrier()

    for b in range(NB):
        pltpu.async_copy(t_sp.at[srcv.at[b]], bufs[b], sems[b])

    def group(m, c):
        for b in range(NB):
            j = m * NB + b
            pltpu.make_async_copy(t_sp.at[srcv.at[0]], bufs[b], sems[b]).wait()
            pltpu.sync_copy(bufs[b], acc_sp.at[dstv.at[j]], add=True)

            @pl.when(m < NG - 1)
            def _():
                pltpu.async_copy(t_sp.at[srcv.at[j + NB]], bufs[b], sems[b])

        return c

    lax.fori_loop(0, NG, group, 0)
    plsc.subcore_barrier()
    pltpu.sync_copy(acc_sp.at[pl.ds(sid * RPT, RPT)],
                    out_hbm.at[cid, pl.ds(sid * RPT, RPT)])


BLK = 5000
GRD = N // BLK


def _tc1_body(x_ref, w1_ref, d0_ref, d1_ref, ydup_ref, dis_ref):
    deg = d0_ref[...] + d1_ref[...] + 1.0
    dis = lax.rsqrt(deg)
    xw = jnp.dot(x_ref[...], w1_ref[...], preferred_element_type=_f32)
    y = xw * dis
    ydup_ref[0] = y
    ydup_ref[1] = y
    dis_ref[...] = dis


def _sigmoid(u):
    return 1.0 / (1.0 + jnp.exp(-u))


def _tc2_body(a0_ref, a1_ref, y_ref, dis_ref, b1_ref, w2_ref, t_ref):
    dis = dis_ref[...]
    z = a0_ref[...] + a1_ref[...] + y_ref[...]
    x1 = _sigmoid(dis * z + b1_ref[...][None, :])
    tcol = jnp.dot(x1, w2_ref[...], preferred_element_type=_f32)
    t_ref[...] = tcol * dis


def _tc3_body(q0_ref, q1_ref, t_ref, dis_ref, b2_ref, o_ref):
    dis = dis_ref[...]
    g = dis * (q0_ref[...] + q1_ref[...] + t_ref[...]) + b2_ref[...]
    o_ref[...] = _sigmoid(g)


def _col_spec(i):
    return pl.BlockSpec((BLK, 1), lambda i: (i, 0))


def kernel(x, pos_edge_index, edge_attr, W1, b1, W2, b2):
    del edge_attr
    src = pos_edge_index[0]
    dst = pos_edge_index[1]
    npad = EPAD - E
    srcp = jnp.concatenate([src, jnp.zeros((npad,), jnp.int32)]).reshape(ECH, CH)
    pad_rows = N + jnp.arange(npad, dtype=jnp.int32) % (NPAD - N)
    dstp = jnp.concatenate([dst, pad_rows]).reshape(ECH, CH)
    srcp2 = srcp + jnp.where(
        jnp.arange(ECH, dtype=jnp.int32)[:, None] >= ECH // 2,
        jnp.int32(N), jnp.int32(0))

    mesh = plsc.VectorSubcoreMesh(core_axis_name="c", subcore_axis_name="s",
                                  num_cores=NC, num_subcores=NS)

    deg_parts = pl.kernel(
        _deg_body,
        out_type=jax.ShapeDtypeStruct((NC, NPAD), _f32),
        mesh=mesh,
        scratch_types=[
            pltpu.VMEM((CPT, CH), jnp.int32),
            pltpu.VMEM((CH,), _f32),
            pltpu.VMEM((CH,), _f32),
            pltpu.VMEM_SHARED((NPAD,), _f32),
        ],
    )(dstp)

    d0 = deg_parts[0, :N, None]
    d1 = deg_parts[1, :N, None]

    ydup, dis = pl.pallas_call(
        _tc1_body,
        grid=(GRD,),
        in_specs=[
            pl.BlockSpec((BLK, D), lambda i: (i, 0)),
            pl.BlockSpec((D, D), lambda i: (0, 0)),
            _col_spec(0),
            _col_spec(0),
        ],
        out_specs=[
            pl.BlockSpec((2, BLK, D), lambda i: (0, i, 0)),
            _col_spec(0),
        ],
        out_shape=[
            jax.ShapeDtypeStruct((2, N, D), _f32),
            jax.ShapeDtypeStruct((N, 1), _f32),
        ],
    )(x, W1, d0, d1)

    y2x = jnp.reshape(ydup, (2 * N, D))

    acc_parts = pl.kernel(
        _row_body,
        out_type=jax.ShapeDtypeStruct((NC, NPAD, D), _f32),
        mesh=mesh,
        scratch_types=[
            pltpu.VMEM((GS, CH), jnp.int32),
            pltpu.VMEM((GS, CH), jnp.int32),
            pltpu.VMEM((GS, CH), jnp.int32),
            pltpu.VMEM((GS, CH), jnp.int32),
            pltpu.VMEM((CH, D), _f32),
            pltpu.VMEM((CH, D), _f32),
            pltpu.SemaphoreType.DMA,
            pltpu.SemaphoreType.DMA,
            pltpu.SemaphoreType.DMA,
            pltpu.SemaphoreType.DMA,
            pltpu.VMEM_SHARED((NPAD, D), _f32),
        ],
    )(y2x, srcp2, dstp)

    t = pl.pallas_call(
        _tc2_body,
        grid=(GRD,),
        in_specs=[
            pl.BlockSpec((BLK, D), lambda i: (i, 0)),
            pl.BlockSpec((BLK, D), lambda i: (i, 0)),
            pl.BlockSpec((BLK, D), lambda i: (i, 0)),
            _col_spec(0),
            pl.BlockSpec((D,), lambda i: (0,)),
            pl.BlockSpec((D, 1), lambda i: (0, 0)),
        ],
        out_specs=_col_spec(0),
        out_shape=jax.ShapeDtypeStruct((N, 1), _f32),
    )(acc_parts[0, :N], acc_parts[1, :N], y2x[:N], dis, b1, W2)

    q_parts = pl.kernel(
        _sca_body,
        out_type=jax.ShapeDtypeStruct((NC, NPAD), _f32),
        mesh=mesh,
        scratch_types=[
            pltpu.VMEM((CPT, CH), jnp.int32),
            pltpu.VMEM((CPT, CH), jnp.int32),
            pltpu.VMEM((CH,), _f32),
            pltpu.VMEM((CH,), _f32),
            pltpu.VMEM((CH,), _f32),
            pltpu.VMEM((CH,), _f32),
            pltpu.SemaphoreType.DMA,
            pltpu.SemaphoreType.DMA,
            pltpu.SemaphoreType.DMA,
            pltpu.SemaphoreType.DMA,
            pltpu.VMEM_SHARED((N,), _f32),
            pltpu.VMEM_SHARED((NPAD,), _f32),
        ],
    )(jnp.reshape(t, (N,)), srcp, dstp)

    out = pl.pallas_call(
        _tc3_body,
        grid=(GRD,),
        in_specs=[
            _col_spec(0),
            _col_spec(0),
            _col_spec(0),
            _col_spec(0),
            pl.BlockSpec((1, 1), lambda i: (0, 0)),
        ],
        out_specs=_col_spec(0),
        out_shape=jax.ShapeDtypeStruct((N, 1), _f32),
    )(q_parts[0, :N, None], q_parts[1, :N, None], t, dis, b2[:, None])

    return out
